# Initial kernel scaffold; baseline (speedup 1.0000x reference)
import jax
import jax.numpy as jnp
from jax.experimental import pallas as pl


def kernel(x, edge_index, W_l1, b_l1, W_r1, W_l2, b_l2, W_r2):
    raise NotImplementedError("write your pallas kernel here")


# speedup vs baseline: 5.7522x; 5.7522x over previous
import functools

import jax
import jax.numpy as jnp
from jax import lax
from jax.experimental import pallas as pl
from jax.experimental.pallas import tpu as pltpu
from jax.experimental.pallas import tpu_sc as plsc

_N = 10000
_E = 320000
_D = 128

_NC = 2
_NS = 16
_NW = _NC * _NS
_C = 128
_NP = 10240
_RPS = _NP // _NS
_ZR = 32

_TOTAL_CHUNKS = _E // _C
_N_FULL = _TOTAL_CHUNKS // _NW
_N_REM = _TOTAL_CHUNKS % _NW

_MESH = dict(core_axis_name="c", subcore_axis_name="s",
             num_cores=_NC, num_subcores=_NS)


def _zero_acc(zrows_hbm, zbuf, acc, row0):
    pltpu.sync_copy(zrows_hbm, zbuf)
    for k in range(_RPS // _ZR):
        pltpu.sync_copy(zbuf, acc.at[pl.ds(row0 + k * _ZR, _ZR)])


def _copy_out(acc, out_hbm, zbuf, row0, out0):
    for k in range(_RPS // _ZR):
        pltpu.sync_copy(acc.at[pl.ds(row0 + k * _ZR, _ZR)], zbuf)
        pltpu.sync_copy(zbuf, out_hbm.at[pl.ds(out0 + k * _ZR, _ZR)])


def _sc_sums_body(x_hbm, src_hbm, dst_hbm, zrows_hbm, sums_hbm,
                  idx_s, idx_d, rows, zbuf, acc, sem):
    cid = lax.axis_index("c")
    sid = lax.axis_index("s")
    w = cid * _NS + sid
    row0 = sid * _RPS

    _zero_acc(zrows_hbm, zbuf, acc, row0)
    plsc.subcore_barrier()

    def process(base):
        pltpu.sync_copy(src_hbm.at[pl.ds(base, _C)], idx_s)
        pltpu.sync_copy(dst_hbm.at[pl.ds(base, _C)], idx_d)
        pltpu.async_copy(x_hbm.at[idx_s], rows, sem).wait()
        pltpu.sync_copy(rows, acc.at[idx_d], add=True)

    @pl.loop(0, _N_FULL)
    def _(g):
        process((w + g * _NW) * _C)

    @pl.when(w < _N_REM)
    def _():
        process((w + _N_FULL * _NW) * _C)

    plsc.subcore_barrier()
    _copy_out(acc, sums_hbm, zbuf, row0, cid * _NP + row0)


def _sc_counts_body(dst_hbm, zrows_hbm, ones_hbm, cnts_hbm,
                    idx_d, ones_v, zbuf, acc, sem):
    cid = lax.axis_index("c")
    sid = lax.axis_index("s")
    w = cid * _NS + sid
    row0 = sid * _RPS

    _zero_acc(zrows_hbm, zbuf, acc, row0)
    pltpu.sync_copy(ones_hbm, ones_v)
    plsc.subcore_barrier()

    def process(base):
        pltpu.sync_copy(dst_hbm.at[pl.ds(base, _C)], idx_d)
        pltpu.sync_copy(ones_v, acc.at[idx_d], add=True)

    @pl.loop(0, _N_FULL)
    def _(g):
        process((w + g * _NW) * _C)

    @pl.when(w < _N_REM)
    def _():
        process((w + _N_FULL * _NW) * _C)

    plsc.subcore_barrier()
    _copy_out(acc, cnts_hbm, zbuf, row0, cid * _NP + row0)


def _sc_sums(t, src, dst, zrows):
    fn = pl.kernel(
        _sc_sums_body,
        out_type=jax.ShapeDtypeStruct((_NC * _NP, _D), jnp.float32),
        mesh=plsc.VectorSubcoreMesh(**_MESH),
        scratch_types=[
            pltpu.VMEM((_C,), jnp.int32),
            pltpu.VMEM((_C,), jnp.int32),
            pltpu.VMEM((_C, _D), jnp.float32),
            pltpu.VMEM((_ZR, _D), jnp.float32),
            pltpu.VMEM_SHARED((_NP, _D), jnp.float32),
            pltpu.SemaphoreType.DMA,
        ])
    return fn(t, src, dst, zrows)


def _sc_counts(dst, zrows, ones):
    fn = pl.kernel(
        _sc_counts_body,
        out_type=jax.ShapeDtypeStruct((_NC * _NP, _D), jnp.float32),
        mesh=plsc.VectorSubcoreMesh(**_MESH),
        scratch_types=[
            pltpu.VMEM((_C,), jnp.int32),
            pltpu.VMEM((_C, _D), jnp.float32),
            pltpu.VMEM((_ZR, _D), jnp.float32),
            pltpu.VMEM_SHARED((_NP, _D), jnp.float32),
            pltpu.SemaphoreType.DMA,
        ])
    return fn(dst, zrows, ones)


def _tc_layer_body(last, sums_ref, cnts_ref, t_ref, wl_ref, bl_ref, wr_ref, o_ref):
    s = sums_ref[0] + sums_ref[1]
    c = cnts_ref[0, :, 0:1] + cnts_ref[1, :, 0:1]
    c = jnp.maximum(c, 1.0)
    aggr = s / c
    z = (jnp.dot(aggr, wl_ref[...], preferred_element_type=jnp.float32)
         + bl_ref[...]
         + jnp.dot(t_ref[...], wr_ref[...], preferred_element_type=jnp.float32))
    if last:
        m = jnp.max(z, axis=-1, keepdims=True)
        e = jnp.exp(z - m)
        lse = jnp.log(jnp.sum(e, axis=-1, keepdims=True)) + m
        o_ref[...] = z - lse
    else:
        o_ref[...] = jnp.maximum(z, 0.0)


def _tc_layer(sums, cnts, t, W_l, b_l, W_r, last):
    R = 1024
    return pl.pallas_call(
        functools.partial(_tc_layer_body, last),
        grid=(_NP // R,),
        in_specs=[
            pl.BlockSpec((_NC, R, _D), lambda i: (0, i, 0)),
            pl.BlockSpec((_NC, R, _D), lambda i: (0, i, 0)),
            pl.BlockSpec((R, _D), lambda i: (i, 0)),
            pl.BlockSpec((_D, _D), lambda i: (0, 0)),
            pl.BlockSpec((1, _D), lambda i: (0, 0)),
            pl.BlockSpec((_D, _D), lambda i: (0, 0)),
        ],
        out_specs=pl.BlockSpec((R, _D), lambda i: (i, 0)),
        out_shape=jax.ShapeDtypeStruct((_NP, _D), jnp.float32),
    )(sums, cnts, t, W_l, b_l, W_r)


def kernel(x, edge_index, W_l1, b_l1, W_r1, W_l2, b_l2, W_r2):
    src = edge_index[0]
    dst = edge_index[1]
    x_pad = jnp.concatenate(
        [x, jnp.zeros((_NP - _N, _D), jnp.float32)], axis=0)
    zrows = jnp.zeros((_ZR, _D), jnp.float32)
    ones = jnp.ones((_C, _D), jnp.float32)
    b1 = b_l1.reshape(1, _D)
    b2 = b_l2.reshape(1, _D)

    cnts = _sc_counts(dst, zrows, ones).reshape(_NC, _NP, _D)
    sums1 = _sc_sums(x, src, dst, zrows).reshape(_NC, _NP, _D)
    h = _tc_layer(sums1, cnts, x_pad, W_l1, b1, W_r1, last=False)

    sums2 = _sc_sums(h, src, dst, zrows).reshape(_NC, _NP, _D)
    out = _tc_layer(sums2, cnts, h, W_l2, b2, W_r2, last=True)
    return out[:_N]


# speedup vs baseline: 7.8270x; 1.3607x over previous
import functools

import jax
import jax.numpy as jnp
from jax import lax
from jax.experimental import pallas as pl
from jax.experimental.pallas import tpu as pltpu
from jax.experimental.pallas import tpu_sc as plsc

_N = 10000
_E = 320000
_D = 128

_NC = 2
_NS = 16
_NW = _NC * _NS
_C = 128
_NP = 10112
_RPS = _NP // _NS

_TOTAL_CHUNKS = _E // _C
_NPW = _TOTAL_CHUNKS // _NW
_REM = _TOTAL_CHUNKS % _NW

_MESH = dict(core_axis_name="c", subcore_axis_name="s",
             num_cores=_NC, num_subcores=_NS)

_PIECES = [(0, _C), (_C, _C), (2 * _C, _C), (3 * _C, _C), (4 * _C, _RPS - 4 * _C)]


def _zero_acc(zsrc_hbm, buf, acc, row0):
    pltpu.sync_copy(zsrc_hbm, buf)
    for off, n in _PIECES:
        pltpu.sync_copy(buf.at[pl.ds(0, n)], acc.at[pl.ds(row0 + off, n)])


def _copy_out(acc, buf, out_hbm, row0, out0):
    for off, n in _PIECES:
        pltpu.sync_copy(acc.at[pl.ds(row0 + off, n)], buf.at[pl.ds(0, n)])
        pltpu.sync_copy(buf.at[pl.ds(0, n)], out_hbm.at[pl.ds(out0 + off, n)])


def _sums_body(x_hbm, src_hbm, dst_hbm, zrows_hbm, sums_hbm,
               is0, is1, id0, id1, r0, r1, acc, s0, s1):
    cid = lax.axis_index("c")
    sid = lax.axis_index("s")
    w = cid * _NS + sid
    row0 = sid * _RPS

    _zero_acc(zrows_hbm, r0, acc, row0)
    plsc.subcore_barrier()

    step = _NW * _C

    @pl.loop(0, _NPW // 2)
    def _(t):
        base = (w + t * 2 * _NW) * _C
        pltpu.sync_copy(src_hbm.at[pl.ds(base, _C)], is0)
        h0 = pltpu.async_copy(x_hbm.at[is0], r0, s0)
        pltpu.sync_copy(src_hbm.at[pl.ds(base + step, _C)], is1)
        h1 = pltpu.async_copy(x_hbm.at[is1], r1, s1)
        pltpu.sync_copy(dst_hbm.at[pl.ds(base, _C)], id0)
        pltpu.sync_copy(dst_hbm.at[pl.ds(base + step, _C)], id1)
        h0.wait()
        pltpu.sync_copy(r0, acc.at[id0], add=True)
        h1.wait()
        pltpu.sync_copy(r1, acc.at[id1], add=True)

    @pl.when(w < _REM)
    def _():
        base = (w + _NPW * _NW) * _C
        pltpu.sync_copy(src_hbm.at[pl.ds(base, _C)], is0)
        h = pltpu.async_copy(x_hbm.at[is0], r0, s0)
        pltpu.sync_copy(dst_hbm.at[pl.ds(base, _C)], id0)
        h.wait()
        pltpu.sync_copy(r0, acc.at[id0], add=True)

    plsc.subcore_barrier()
    _copy_out(acc, r0, sums_hbm, row0, cid * _NP + row0)


def _sc_sums(t, src, dst, zrows):
    fn = pl.kernel(
        _sums_body,
        out_type=jax.ShapeDtypeStruct((_NC * _NP, _D), jnp.float32),
        mesh=plsc.VectorSubcoreMesh(**_MESH),
        scratch_types=[
            pltpu.VMEM((_C,), jnp.int32),
            pltpu.VMEM((_C,), jnp.int32),
            pltpu.VMEM((_C,), jnp.int32),
            pltpu.VMEM((_C,), jnp.int32),
            pltpu.VMEM((_C, _D), jnp.float32),
            pltpu.VMEM((_C, _D), jnp.float32),
            pltpu.VMEM_SHARED((_NP, _D), jnp.float32),
            pltpu.SemaphoreType.DMA,
            pltpu.SemaphoreType.DMA,
        ])
    return fn(t, src, dst, zrows)


def _cnts_body(dst_hbm, zrows_hbm, ones_hbm, cnts_hbm,
               idv, ones_v, acc):
    cid = lax.axis_index("c")
    sid = lax.axis_index("s")
    w = cid * _NS + sid
    row0 = sid * _RPS

    _zero_acc(zrows_hbm, ones_v, acc, row0)
    pltpu.sync_copy(ones_hbm, ones_v)
    plsc.subcore_barrier()

    def chunk(base):
        pltpu.sync_copy(dst_hbm.at[pl.ds(base, _C)], idv)
        pltpu.sync_copy(ones_v, acc.at[idv], add=True)

    @pl.loop(0, _NPW)
    def _(j):
        chunk((w + j * _NW) * _C)

    @pl.when(w < _REM)
    def _():
        chunk((w + _NPW * _NW) * _C)

    plsc.subcore_barrier()
    _copy_out(acc, ones_v, cnts_hbm, row0, cid * _NP + row0)


def _sc_cnts(dst, zrows, ones):
    fn = pl.kernel(
        _cnts_body,
        out_type=jax.ShapeDtypeStruct((_NC * _NP, _D), jnp.float32),
        mesh=plsc.VectorSubcoreMesh(**_MESH),
        scratch_types=[
            pltpu.VMEM((_C,), jnp.int32),
            pltpu.VMEM((_C, _D), jnp.float32),
            pltpu.VMEM_SHARED((_NP, _D), jnp.float32),
        ])
    return fn(dst, zrows, ones)


def _tc_layer_body(last, sums_ref, cnts_ref, t_ref, wl_ref, bl_ref, wr_ref, o_ref):
    s = sums_ref[0] + sums_ref[1]
    c = cnts_ref[0, :, 0:1] + cnts_ref[1, :, 0:1]
    c = jnp.maximum(c, 1.0)
    aggr = s / c
    z = (jnp.dot(aggr, wl_ref[...], preferred_element_type=jnp.float32)
         + bl_ref[...]
         + jnp.dot(t_ref[...], wr_ref[...], preferred_element_type=jnp.float32))
    if last:
        m = jnp.max(z, axis=-1, keepdims=True)
        e = jnp.exp(z - m)
        lse = jnp.log(jnp.sum(e, axis=-1, keepdims=True)) + m
        o_ref[...] = z - lse
    else:
        o_ref[...] = jnp.maximum(z, 0.0)


def _tc_layer(sums, cnts, t, W_l, b_l, W_r, last):
    R = 1264
    return pl.pallas_call(
        functools.partial(_tc_layer_body, last),
        grid=(_NP // R,),
        in_specs=[
            pl.BlockSpec((_NC, R, _D), lambda i: (0, i, 0)),
            pl.BlockSpec((_NC, R, _D), lambda i: (0, i, 0)),
            pl.BlockSpec((R, _D), lambda i: (i, 0)),
            pl.BlockSpec((_D, _D), lambda i: (0, 0)),
            pl.BlockSpec((1, _D), lambda i: (0, 0)),
            pl.BlockSpec((_D, _D), lambda i: (0, 0)),
        ],
        out_specs=pl.BlockSpec((R, _D), lambda i: (i, 0)),
        out_shape=jax.ShapeDtypeStruct((_NP, _D), jnp.float32),
    )(sums, cnts, t, W_l, b_l, W_r)


def kernel(x, edge_index, W_l1, b_l1, W_r1, W_l2, b_l2, W_r2):
    src = edge_index[0]
    dst = edge_index[1]
    x_pad = jnp.concatenate(
        [x, jnp.zeros((_NP - _N, _D), jnp.float32)], axis=0)
    zrows = jnp.zeros((_C, _D), jnp.float32)
    ones = jnp.ones((_C, _D), jnp.float32)
    b1 = b_l1.reshape(1, _D)
    b2 = b_l2.reshape(1, _D)

    cnts = _sc_cnts(dst, zrows, ones).reshape(_NC, _NP, _D)
    sums1 = _sc_sums(x, src, dst, zrows).reshape(_NC, _NP, _D)
    h = _tc_layer(sums1, cnts, x_pad, W_l1, b1, W_r1, last=False)

    sums2 = _sc_sums(h, src, dst, zrows).reshape(_NC, _NP, _D)
    out = _tc_layer(sums2, cnts, h, W_l2, b2, W_r2, last=True)
    return out[:_N]


# speedup vs baseline: 8.1494x; 1.0412x over previous
import functools

import jax
import jax.numpy as jnp
from jax import lax
from jax.experimental import pallas as pl
from jax.experimental.pallas import tpu as pltpu
from jax.experimental.pallas import tpu_sc as plsc

_N = 10000
_E = 320000
_D = 128

_NC = 2
_NS = 16
_NW = _NC * _NS
_C = 128
_CW = 16
_NP = 10112
_RPS = _NP // _NS

_TOTAL_CHUNKS = _E // _C
_NPW = _TOTAL_CHUNKS // _NW
_REM = _TOTAL_CHUNKS % _NW

_MESH = dict(core_axis_name="c", subcore_axis_name="s",
             num_cores=_NC, num_subcores=_NS)

_PIECES = [(0, _C), (_C, _C), (2 * _C, _C), (3 * _C, _C), (4 * _C, _RPS - 4 * _C)]


def _zero_acc(zsrc_hbm, buf, acc, row0):
    pltpu.sync_copy(zsrc_hbm, buf)
    for off, n in _PIECES:
        pltpu.sync_copy(buf.at[pl.ds(0, n)], acc.at[pl.ds(row0 + off, n)])


def _copy_out(acc, buf, out_hbm, row0, out0):
    for off, n in _PIECES:
        pltpu.sync_copy(acc.at[pl.ds(row0 + off, n)], buf.at[pl.ds(0, n)])
        pltpu.sync_copy(buf.at[pl.ds(0, n)], out_hbm.at[pl.ds(out0 + off, n)])


def _sums_body(x_hbm, src_hbm, dst_hbm, zrows_hbm, sums_hbm,
               is0, is1, id0, id1, r0, r1, acc, s0, s1):
    cid = lax.axis_index("c")
    sid = lax.axis_index("s")
    w = cid * _NS + sid
    row0 = sid * _RPS

    _zero_acc(zrows_hbm, r0, acc, row0)
    plsc.subcore_barrier()

    step = _NW * _C

    @pl.loop(0, _NPW // 2)
    def _(t):
        base = (w + t * 2 * _NW) * _C
        pltpu.sync_copy(src_hbm.at[pl.ds(base, _C)], is0)
        h0 = pltpu.async_copy(x_hbm.at[is0], r0, s0)
        pltpu.sync_copy(src_hbm.at[pl.ds(base + step, _C)], is1)
        h1 = pltpu.async_copy(x_hbm.at[is1], r1, s1)
        pltpu.sync_copy(dst_hbm.at[pl.ds(base, _C)], id0)
        pltpu.sync_copy(dst_hbm.at[pl.ds(base + step, _C)], id1)
        h0.wait()
        pltpu.sync_copy(r0, acc.at[id0], add=True)
        h1.wait()
        pltpu.sync_copy(r1, acc.at[id1], add=True)

    @pl.when(w < _REM)
    def _():
        base = (w + _NPW * _NW) * _C
        pltpu.sync_copy(src_hbm.at[pl.ds(base, _C)], is0)
        h = pltpu.async_copy(x_hbm.at[is0], r0, s0)
        pltpu.sync_copy(dst_hbm.at[pl.ds(base, _C)], id0)
        h.wait()
        pltpu.sync_copy(r0, acc.at[id0], add=True)

    plsc.subcore_barrier()
    _copy_out(acc, r0, sums_hbm, row0, cid * _NP + row0)


def _sc_sums(t, src, dst, zrows):
    fn = pl.kernel(
        _sums_body,
        out_type=jax.ShapeDtypeStruct((_NC * _NP, _D), jnp.float32),
        mesh=plsc.VectorSubcoreMesh(**_MESH),
        scratch_types=[
            pltpu.VMEM((_C,), jnp.int32),
            pltpu.VMEM((_C,), jnp.int32),
            pltpu.VMEM((_C,), jnp.int32),
            pltpu.VMEM((_C,), jnp.int32),
            pltpu.VMEM((_C, _D), jnp.float32),
            pltpu.VMEM((_C, _D), jnp.float32),
            pltpu.VMEM_SHARED((_NP, _D), jnp.float32),
            pltpu.SemaphoreType.DMA,
            pltpu.SemaphoreType.DMA,
        ])
    return fn(t, src, dst, zrows)


def _cnts_body(dst_hbm, zrows_hbm, ones_hbm, cnts_hbm,
               id0, id1, ones_v, acc, s0, s1):
    cid = lax.axis_index("c")
    sid = lax.axis_index("s")
    w = cid * _NS + sid
    row0 = sid * _RPS

    _zero_acc(zrows_hbm, ones_v, acc, row0)
    pltpu.sync_copy(ones_hbm, ones_v)
    plsc.subcore_barrier()

    step = _NW * _C

    @pl.loop(0, _NPW // 2)
    def _(t):
        base = (w + t * 2 * _NW) * _C
        h0 = pltpu.async_copy(dst_hbm.at[pl.ds(base, _C)], id0, s0)
        h1 = pltpu.async_copy(dst_hbm.at[pl.ds(base + step, _C)], id1, s1)
        h0.wait()
        pltpu.sync_copy(ones_v, acc.at[id0], add=True)
        h1.wait()
        pltpu.sync_copy(ones_v, acc.at[id1], add=True)

    @pl.when(w < _REM)
    def _():
        base = (w + _NPW * _NW) * _C
        pltpu.sync_copy(dst_hbm.at[pl.ds(base, _C)], id0)
        pltpu.sync_copy(ones_v, acc.at[id0], add=True)

    plsc.subcore_barrier()
    _copy_out(acc, ones_v, cnts_hbm, row0, cid * _NP + row0)


def _sc_cnts(dst, zrows, ones):
    fn = pl.kernel(
        _cnts_body,
        out_type=jax.ShapeDtypeStruct((_NC * _NP, _D), jnp.float32),
        mesh=plsc.VectorSubcoreMesh(**_MESH),
        scratch_types=[
            pltpu.VMEM((_C,), jnp.int32),
            pltpu.VMEM((_C,), jnp.int32),
            pltpu.VMEM((_C, _D), jnp.float32),
            pltpu.VMEM_SHARED((_NP, _D), jnp.float32),
            pltpu.SemaphoreType.DMA,
            pltpu.SemaphoreType.DMA,
        ])
    return fn(dst, zrows, ones)


def _tc_layer_body(last, sums_ref, cnts_ref, t_ref, wl_ref, bl_ref, wr_ref, o_ref):
    s = sums_ref[0] + sums_ref[1]
    c = cnts_ref[0, :, 0:1] + cnts_ref[1, :, 0:1]
    c = jnp.maximum(c, 1.0)
    aggr = s / c
    z = (jnp.dot(aggr, wl_ref[...], preferred_element_type=jnp.float32)
         + bl_ref[...]
         + jnp.dot(t_ref[...], wr_ref[...], preferred_element_type=jnp.float32))
    if last:
        m = jnp.max(z, axis=-1, keepdims=True)
        e = jnp.exp(z - m)
        lse = jnp.log(jnp.sum(e, axis=-1, keepdims=True)) + m
        o_ref[...] = z - lse
    else:
        o_ref[...] = jnp.maximum(z, 0.0)


def _tc_layer(sums, cnts, t, W_l, b_l, W_r, last):
    R = 1264
    return pl.pallas_call(
        functools.partial(_tc_layer_body, last),
        grid=(_NP // R,),
        in_specs=[
            pl.BlockSpec((_NC, R, _D), lambda i: (0, i, 0)),
            pl.BlockSpec((_NC, R, _D), lambda i: (0, i, 0)),
            pl.BlockSpec((R, _D), lambda i: (i, 0)),
            pl.BlockSpec((_D, _D), lambda i: (0, 0)),
            pl.BlockSpec((1, _D), lambda i: (0, 0)),
            pl.BlockSpec((_D, _D), lambda i: (0, 0)),
        ],
        out_specs=pl.BlockSpec((R, _D), lambda i: (i, 0)),
        out_shape=jax.ShapeDtypeStruct((_NP, _D), jnp.float32),
    )(sums, cnts, t, W_l, b_l, W_r)


def kernel(x, edge_index, W_l1, b_l1, W_r1, W_l2, b_l2, W_r2):
    src = edge_index[0]
    dst = edge_index[1]
    x_pad = jnp.concatenate(
        [x, jnp.zeros((_NP - _N, _D), jnp.float32)], axis=0)
    zrows = jnp.zeros((_C, _D), jnp.float32)
    ones = jnp.ones((_C, _D), jnp.float32)
    b1 = b_l1.reshape(1, _D)
    b2 = b_l2.reshape(1, _D)

    cnts = _sc_cnts(dst, zrows, ones).reshape(_NC, _NP, _D)
    sums1 = _sc_sums(x, src, dst, zrows).reshape(_NC, _NP, _D)
    h = _tc_layer(sums1, cnts, x_pad, W_l1, b1, W_r1, last=False)

    sums2 = _sc_sums(h, src, dst, zrows).reshape(_NC, _NP, _D)
    out = _tc_layer(sums2, cnts, h, W_l2, b2, W_r2, last=True)
    return out[:_N]
